# baseline (speedup 1.0000x reference)
import functools

import jax
import jax.numpy as jnp
from jax import lax
from jax.experimental import pallas as pl
from jax.experimental.pallas import tpu as pltpu
from jax.experimental.pallas import tpu_sc as plsc

N = 10000
D = 128
E = 320000
NC = 2
NS = 16
EPW = E // (NC * NS)
CHUNK = 64
EPAD = 10240
NCHUNK = EPAD // CHUNK
NPAD = 10016
DUMMY = 10008
S0 = 624
LAST_OFF = (NS - 1) * S0
LAST_LEN = N - LAST_OFF
ZLAST_LEN = NPAD - LAST_OFF

DW = 16

_mesh = plsc.VectorSubcoreMesh(core_axis_name="c", subcore_axis_name="s")


@functools.partial(
    pl.kernel,
    mesh=_mesh,
    out_type=jax.ShapeDtypeStruct((NC * NS * N,), jnp.float32),
    scratch_types=[
        pltpu.VMEM((EPW,), jnp.int32),
        pltpu.VMEM((N,), jnp.float32),
    ],
    compiler_params=pltpu.CompilerParams(needs_layout_passes=False),
)
def _deg_kernel(dst_hbm, out_hbm, idxv, hist):
    c = lax.axis_index("c")
    s = lax.axis_index("s")
    wid = c * NS + s

    def zbody(i, carry):
        hist[pl.ds(i * 16, 16)] = jnp.zeros((16,), jnp.float32)
        return carry

    lax.fori_loop(0, N // 16, zbody, 0)
    pltpu.sync_copy(dst_hbm.at[pl.ds(wid * EPW, EPW)], idxv)

    def body(i, carry):
        idx = idxv[pl.ds(i * 16, 16)]
        plsc.addupdate_scatter(hist, [idx], jnp.ones((16,), jnp.float32))
        return carry

    lax.fori_loop(0, EPW // 16, body, 0)
    pltpu.sync_copy(hist, out_hbm.at[pl.ds(wid * N, N)])


@functools.partial(
    pl.kernel,
    mesh=_mesh,
    out_type=jax.ShapeDtypeStruct((NC * N, D), jnp.float32),
    scratch_types=[
        pltpu.VMEM((2, CHUNK), jnp.int32),
        pltpu.VMEM((2, CHUNK), jnp.int32),
        pltpu.VMEM((CHUNK, D), jnp.float32),
        pltpu.VMEM((CHUNK, D), jnp.float32),
        pltpu.VMEM_SHARED((NPAD, D), jnp.float32),
        pltpu.SemaphoreType.DMA,
        pltpu.SemaphoreType.DMA,
        pltpu.SemaphoreType.DMA,
    ],
)
def _agg_kernel(h_hbm, idx_hbm, zeros_hbm, out_hbm,
                buf0, buf1, rows0, rows1, acc, semI1, semG0, semG1):
    c = lax.axis_index("c")
    s = lax.axis_index("s")
    wid = c * NS + s

    @pl.when(s < NS - 1)
    def _():
        pltpu.sync_copy(zeros_hbm.at[pl.ds(0, S0)], acc.at[pl.ds(s * S0, S0)])

    @pl.when(s == NS - 1)
    def _():
        pltpu.sync_copy(zeros_hbm, acc.at[pl.ds(LAST_OFF, ZLAST_LEN)])

    plsc.subcore_barrier()
    pltpu.sync_copy(idx_hbm.at[wid, 0], buf0)
    pltpu.async_copy(idx_hbm.at[wid, 1], buf1, semI1)
    pltpu.async_copy(h_hbm.at[buf0.at[0]], rows0, semG0)

    def body(j, carry):
        i0 = 2 * j
        pltpu.make_async_copy(idx_hbm.at[wid, i0 + 1], buf1, semI1).wait()
        pltpu.async_copy(h_hbm.at[buf1.at[0]], rows1, semG1)
        pltpu.make_async_copy(h_hbm.at[buf0.at[0]], rows0, semG0).wait()
        pltpu.sync_copy(rows0, acc.at[buf0.at[1]], add=True)

        @pl.when(j < NCHUNK // 2 - 1)
        def _():
            pltpu.sync_copy(idx_hbm.at[wid, i0 + 2], buf0)
            pltpu.async_copy(h_hbm.at[buf0.at[0]], rows0, semG0)

        pltpu.make_async_copy(h_hbm.at[buf1.at[0]], rows1, semG1).wait()
        pltpu.sync_copy(rows1, acc.at[buf1.at[1]], add=True)

        @pl.when(j < NCHUNK // 2 - 1)
        def _():
            pltpu.async_copy(idx_hbm.at[wid, i0 + 3], buf1, semI1)

        return carry

    lax.fori_loop(0, NCHUNK // 2, body, 0)
    plsc.subcore_barrier()

    @pl.when(s < NS - 1)
    def _():
        pltpu.sync_copy(acc.at[pl.ds(s * S0, S0)],
                        out_hbm.at[pl.ds(c * N + s * S0, S0)])

    @pl.when(s == NS - 1)
    def _():
        pltpu.sync_copy(acc.at[pl.ds(LAST_OFF, LAST_LEN)],
                        out_hbm.at[pl.ds(c * N + LAST_OFF, LAST_LEN)])


BT = 1000


def _tc1_body(x_ref, w_ref, b_ref, degp_ref, h1p_ref, self1_ref, dis_ref):
    deg = jnp.sum(degp_ref[...], axis=0) + 1.0
    dis = lax.rsqrt(deg)
    h = jnp.dot(x_ref[...], w_ref[...], preferred_element_type=jnp.float32)
    h1p_ref[...] = h * dis
    self1_ref[...] = h * (dis * dis) + b_ref[...]
    dis_ref[...] = dis


_tc1 = pl.pallas_call(
    _tc1_body,
    grid=(N // BT,),
    in_specs=[
        pl.BlockSpec((BT, D), lambda i: (i, 0)),
        pl.BlockSpec((D, D), lambda i: (0, 0)),
        pl.BlockSpec((1, D), lambda i: (0, 0)),
        pl.BlockSpec((NC * NS, BT, 1), lambda i: (0, i, 0)),
    ],
    out_specs=[
        pl.BlockSpec((BT, D), lambda i: (i, 0)),
        pl.BlockSpec((BT, D), lambda i: (i, 0)),
        pl.BlockSpec((BT, 1), lambda i: (i, 0)),
    ],
    out_shape=[
        jax.ShapeDtypeStruct((N, D), jnp.float32),
        jax.ShapeDtypeStruct((N, D), jnp.float32),
        jax.ShapeDtypeStruct((N, 1), jnp.float32),
    ],
)


def _tc2_body(aggp_ref, self1_ref, dis_ref, w_ref, b_ref, h2p_ref, self2_ref):
    dis = dis_ref[...]
    y1 = jnp.maximum((aggp_ref[0] + aggp_ref[1]) * dis + self1_ref[...], 0.0)
    h2 = jnp.dot(y1, w_ref[...], preferred_element_type=jnp.float32)
    h2p_ref[...] = h2 * dis
    self2_ref[...] = h2 * (dis * dis) + b_ref[...]


_tc2 = pl.pallas_call(
    _tc2_body,
    grid=(N // BT,),
    in_specs=[
        pl.BlockSpec((2, BT, D), lambda i: (0, i, 0)),
        pl.BlockSpec((BT, D), lambda i: (i, 0)),
        pl.BlockSpec((BT, 1), lambda i: (i, 0)),
        pl.BlockSpec((D, D), lambda i: (0, 0)),
        pl.BlockSpec((1, D), lambda i: (0, 0)),
    ],
    out_specs=[
        pl.BlockSpec((BT, D), lambda i: (i, 0)),
        pl.BlockSpec((BT, D), lambda i: (i, 0)),
    ],
    out_shape=[
        jax.ShapeDtypeStruct((N, D), jnp.float32),
        jax.ShapeDtypeStruct((N, D), jnp.float32),
    ],
)


def _tc3_body(aggp_ref, self2_ref, dis_ref, out_ref):
    out_ref[...] = jnp.maximum(
        (aggp_ref[0] + aggp_ref[1]) * dis_ref[...] + self2_ref[...], 0.0)


_tc3 = pl.pallas_call(
    _tc3_body,
    grid=(N // BT,),
    in_specs=[
        pl.BlockSpec((2, BT, D), lambda i: (0, i, 0)),
        pl.BlockSpec((BT, D), lambda i: (i, 0)),
        pl.BlockSpec((BT, 1), lambda i: (i, 0)),
    ],
    out_specs=pl.BlockSpec((BT, D), lambda i: (i, 0)),
    out_shape=jax.ShapeDtypeStruct((N, D), jnp.float32),
)


def kernel(x, edge_index, W1, b1, W2, b2):
    ei = edge_index.astype(jnp.int32)
    NW = NC * NS
    src2 = ei[0].reshape(NW, EPW)
    dst2 = ei[1].reshape(NW, EPW)
    srcp = jnp.pad(src2, ((0, 0), (0, EPAD - EPW)))
    dstp = jnp.pad(dst2, ((0, 0), (0, EPAD - EPW)), constant_values=DUMMY)
    idxcat = jnp.stack([srcp.reshape(NW, NCHUNK, CHUNK),
                        dstp.reshape(NW, NCHUNK, CHUNK)], axis=2)
    dst_flat = ei[1]
    zeros_r = jnp.zeros((ZLAST_LEN, D), jnp.float32)

    degp = _deg_kernel(dst_flat).reshape(NC * NS, N, 1)
    h1p, self1, dis = _tc1(x, W1, b1[None, :], degp)
    agg1 = _agg_kernel(h1p, idxcat, zeros_r).reshape(NC, N, D)
    h2p, self2 = _tc2(agg1, self1, dis, W2, b2[None, :])
    agg2 = _agg_kernel(h2p, idxcat, zeros_r).reshape(NC, N, D)
    return _tc3(agg2, self2, dis)


# speedup vs baseline: 1.6859x; 1.6859x over previous
import functools

import jax
import jax.numpy as jnp
from jax import lax
from jax.experimental import pallas as pl
from jax.experimental.pallas import tpu as pltpu
from jax.experimental.pallas import tpu_sc as plsc

N = 10000
D = 128
E = 320000
NC = 2
NS = 16
EPW = E // (NC * NS)
CHUNK = 80
EPAD = 10000
NCHUNK = EPAD // CHUNK
NPAD = 10000
DUMMY = 0
S0 = 624
LAST_OFF = (NS - 1) * S0
LAST_LEN = N - LAST_OFF
ZLAST_LEN = NPAD - LAST_OFF

DW = 16

_mesh = plsc.VectorSubcoreMesh(core_axis_name="c", subcore_axis_name="s")


@functools.partial(
    pl.kernel,
    mesh=_mesh,
    out_type=jax.ShapeDtypeStruct((NC * NS * N,), jnp.float32),
    scratch_types=[
        pltpu.VMEM((EPW,), jnp.int32),
        pltpu.VMEM((N,), jnp.float32),
    ],
    compiler_params=pltpu.CompilerParams(needs_layout_passes=False),
)
def _deg_kernel(dst_hbm, out_hbm, idxv, hist):
    c = lax.axis_index("c")
    s = lax.axis_index("s")
    wid = c * NS + s

    def zbody(i, carry):
        hist[pl.ds(i * 16, 16)] = jnp.zeros((16,), jnp.float32)
        return carry

    lax.fori_loop(0, N // 16, zbody, 0)
    pltpu.sync_copy(dst_hbm.at[pl.ds(wid * EPW, EPW)], idxv)

    def body(i, carry):
        idx = idxv[pl.ds(i * 16, 16)]
        plsc.addupdate_scatter(hist, [idx], jnp.ones((16,), jnp.float32))
        return carry

    lax.fori_loop(0, EPW // 16, body, 0)
    pltpu.sync_copy(hist, out_hbm.at[pl.ds(wid * N, N)])


@functools.partial(
    pl.kernel,
    mesh=_mesh,
    out_type=jax.ShapeDtypeStruct((NC * N, D), jnp.float32),
    scratch_types=[
        pltpu.VMEM((2, CHUNK), jnp.int32),
        pltpu.VMEM((2, CHUNK), jnp.int32),
        pltpu.VMEM((CHUNK, D), jnp.float32),
        pltpu.VMEM_SHARED((NPAD, D), jnp.float32),
        pltpu.SemaphoreType.DMA,
        pltpu.SemaphoreType.DMA,
        pltpu.SemaphoreType.DMA,
    ],
)
def _agg_kernel(h_hbm, idx_hbm, zeros_hbm, out_hbm,
                buf0, buf1, rows0, acc, semI0, semI1, semG0):
    c = lax.axis_index("c")
    s = lax.axis_index("s")
    wid = c * NS + s

    @pl.when(s < NS - 1)
    def _():
        pltpu.sync_copy(zeros_hbm.at[pl.ds(0, S0)], acc.at[pl.ds(s * S0, S0)])

    @pl.when(s == NS - 1)
    def _():
        pltpu.sync_copy(zeros_hbm, acc.at[pl.ds(LAST_OFF, ZLAST_LEN)])

    plsc.subcore_barrier()
    pltpu.sync_copy(idx_hbm.at[wid, 0], buf0)

    def body(j, carry):
        i0 = 2 * j
        pltpu.async_copy(idx_hbm.at[wid, i0 + 1], buf1, semI1)
        pltpu.async_copy(h_hbm.at[buf0.at[0]], rows0, semG0).wait()
        pltpu.sync_copy(rows0, acc.at[buf0.at[1]], add=True)
        pltpu.make_async_copy(idx_hbm.at[wid, i0 + 1], buf1, semI1).wait()
        pltpu.async_copy(idx_hbm.at[wid, i0 + 2], buf0, semI0)
        pltpu.async_copy(h_hbm.at[buf1.at[0]], rows0, semG0).wait()
        pltpu.sync_copy(rows0, acc.at[buf1.at[1]], add=True)
        pltpu.make_async_copy(idx_hbm.at[wid, i0 + 2], buf0, semI0).wait()
        return carry

    lax.fori_loop(0, (NCHUNK - 1) // 2, body, 0)
    pltpu.async_copy(h_hbm.at[buf0.at[0]], rows0, semG0).wait()
    pltpu.sync_copy(rows0, acc.at[buf0.at[1]], add=True)
    plsc.subcore_barrier()

    @pl.when(s < NS - 1)
    def _():
        pltpu.sync_copy(acc.at[pl.ds(s * S0, S0)],
                        out_hbm.at[pl.ds(c * N + s * S0, S0)])

    @pl.when(s == NS - 1)
    def _():
        pltpu.sync_copy(acc.at[pl.ds(LAST_OFF, LAST_LEN)],
                        out_hbm.at[pl.ds(c * N + LAST_OFF, LAST_LEN)])


BT = 1000


def _tc1_body(x_ref, w_ref, b_ref, degp_ref, h1p_ref, self1_ref, dis_ref):
    deg = jnp.sum(degp_ref[...], axis=0) + 1.0
    dis = lax.rsqrt(deg)
    h = jnp.dot(x_ref[...], w_ref[...], preferred_element_type=jnp.float32)
    h1p_ref[...] = h * dis
    self1_ref[...] = h * (dis * dis) + b_ref[...]
    dis_ref[...] = dis


_tc1 = pl.pallas_call(
    _tc1_body,
    grid=(N // BT,),
    in_specs=[
        pl.BlockSpec((BT, D), lambda i: (i, 0)),
        pl.BlockSpec((D, D), lambda i: (0, 0)),
        pl.BlockSpec((1, D), lambda i: (0, 0)),
        pl.BlockSpec((NC * NS, BT, 1), lambda i: (0, i, 0)),
    ],
    out_specs=[
        pl.BlockSpec((BT, D), lambda i: (i, 0)),
        pl.BlockSpec((BT, D), lambda i: (i, 0)),
        pl.BlockSpec((BT, 1), lambda i: (i, 0)),
    ],
    out_shape=[
        jax.ShapeDtypeStruct((N, D), jnp.float32),
        jax.ShapeDtypeStruct((N, D), jnp.float32),
        jax.ShapeDtypeStruct((N, 1), jnp.float32),
    ],
)


def _tc2_body(aggp_ref, self1_ref, dis_ref, w_ref, b_ref, h2p_ref, self2_ref):
    dis = dis_ref[...]
    y1 = jnp.maximum((aggp_ref[0] + aggp_ref[1]) * dis + self1_ref[...], 0.0)
    h2 = jnp.dot(y1, w_ref[...], preferred_element_type=jnp.float32)
    h2p_ref[...] = h2 * dis
    self2_ref[...] = h2 * (dis * dis) + b_ref[...]


_tc2 = pl.pallas_call(
    _tc2_body,
    grid=(N // BT,),
    in_specs=[
        pl.BlockSpec((2, BT, D), lambda i: (0, i, 0)),
        pl.BlockSpec((BT, D), lambda i: (i, 0)),
        pl.BlockSpec((BT, 1), lambda i: (i, 0)),
        pl.BlockSpec((D, D), lambda i: (0, 0)),
        pl.BlockSpec((1, D), lambda i: (0, 0)),
    ],
    out_specs=[
        pl.BlockSpec((BT, D), lambda i: (i, 0)),
        pl.BlockSpec((BT, D), lambda i: (i, 0)),
    ],
    out_shape=[
        jax.ShapeDtypeStruct((N, D), jnp.float32),
        jax.ShapeDtypeStruct((N, D), jnp.float32),
    ],
)


def _tc3_body(aggp_ref, self2_ref, dis_ref, out_ref):
    out_ref[...] = jnp.maximum(
        (aggp_ref[0] + aggp_ref[1]) * dis_ref[...] + self2_ref[...], 0.0)


_tc3 = pl.pallas_call(
    _tc3_body,
    grid=(N // BT,),
    in_specs=[
        pl.BlockSpec((2, BT, D), lambda i: (0, i, 0)),
        pl.BlockSpec((BT, D), lambda i: (i, 0)),
        pl.BlockSpec((BT, 1), lambda i: (i, 0)),
    ],
    out_specs=pl.BlockSpec((BT, D), lambda i: (i, 0)),
    out_shape=jax.ShapeDtypeStruct((N, D), jnp.float32),
)


def kernel(x, edge_index, W1, b1, W2, b2):
    ei = edge_index.astype(jnp.int32)
    NW = NC * NS
    src2 = ei[0].reshape(NW, EPW)
    dst2 = ei[1].reshape(NW, EPW)
    srcp = jnp.pad(src2, ((0, 0), (0, EPAD - EPW)))
    dstp = jnp.pad(dst2, ((0, 0), (0, EPAD - EPW)), constant_values=DUMMY)
    idxcat = jnp.stack([srcp.reshape(NW, NCHUNK, CHUNK),
                        dstp.reshape(NW, NCHUNK, CHUNK)], axis=2)
    dst_flat = ei[1]
    zeros_r = jnp.zeros((ZLAST_LEN, D), jnp.float32)

    degp = _deg_kernel(dst_flat).reshape(NC * NS, N, 1)
    h1p, self1, dis = _tc1(x, W1, b1[None, :], degp)
    agg1 = _agg_kernel(h1p, idxcat, zeros_r).reshape(NC, N, D)
    h2p, self2 = _tc2(agg1, self1, dis, W2, b2[None, :])
    agg2 = _agg_kernel(h2p, idxcat, zeros_r).reshape(NC, N, D)
    return _tc3(agg2, self2, dis)
